# baseline (speedup 1.0000x reference)
import functools

import jax
import jax.numpy as jnp
from jax import lax
from jax.experimental import pallas as pl
from jax.experimental.pallas import tpu as pltpu
from jax.experimental.pallas import tpu_sc as plsc

N = 10000
E = 320000
D = 128
G = 64

NC = 2
NS = 16
NW = NC * NS
CHUNK = 128
EP = NW * 10240
EPW = EP // NW
NCHUNK = EPW // CHUNK
NP = 10240
RPS = NP // NS
ZCH = 128
NZ = RPS // ZCH

ROWS = 2000
NB = N // ROWS


SB = 16
SROWS = SB + 8
NSB = NCHUNK // SB


def _sc_agg_body(h_hbm, src_hbm, dst_hbm, zeros_hbm, out_hbm,
                 sidx0, sidx1, didx0, didx1, rows0, rows1, acc_sh,
                 isem, gsem0, gsem1, ssem0, ssem1, wsem):
    c = lax.axis_index("c")
    s = lax.axis_index("s")
    w = s * NC + c
    wrow = w * NCHUNK

    pltpu.sync_copy(zeros_hbm, rows0)
    for j in range(NZ):
        pltpu.async_copy(rows0, acc_sh.at[pl.ds(s * RPS + j * ZCH, ZCH)], wsem)
    pltpu.sync_copy(src_hbm.at[pl.ds(wrow, SROWS)], sidx0)
    pltpu.sync_copy(dst_hbm.at[pl.ds(wrow, SB)], didx0)
    for j in range(NZ):
        pltpu.make_async_copy(rows0, acc_sh.at[pl.ds(0, ZCH)], wsem).wait()
    pltpu.async_copy(h_hbm.at[sidx0.at[0]], rows0, gsem0)
    pltpu.async_copy(h_hbm.at[sidx0.at[1]], rows1, gsem1)
    plsc.subcore_barrier()

    sb = (sidx0, sidx1)
    db = (didx0, didx1)

    def _make_pair(sbp, dbp):
        def pair(t, _):
            l0 = 2 * t
            l1 = l0 + 1
            pltpu.make_async_copy(h_hbm.at[sbp.at[0]], rows0, gsem0).wait()
            pltpu.async_copy(rows0, acc_sh.at[dbp.at[l0]], ssem0, add=True)
            pltpu.make_async_copy(h_hbm.at[sbp.at[0]], rows1, gsem1).wait()
            pltpu.async_copy(rows1, acc_sh.at[dbp.at[l1]], ssem1, add=True)
            pltpu.make_async_copy(rows0, acc_sh.at[dbp.at[l0]], ssem0).wait()
            pltpu.async_copy(h_hbm.at[sbp.at[l0 + 2]], rows0, gsem0)
            pltpu.make_async_copy(rows1, acc_sh.at[dbp.at[l1]], ssem1).wait()
            pltpu.async_copy(h_hbm.at[sbp.at[l1 + 2]], rows1, gsem1)
            return ()
        return pair

    for jb in range(NSB):
        p = jb % 2
        if jb > 0:
            pltpu.make_async_copy(src_hbm.at[pl.ds(0, SROWS)], sb[p], isem).wait()
            pltpu.make_async_copy(dst_hbm.at[pl.ds(0, SB)], db[p], isem).wait()
        pair = _make_pair(sb[p], db[p])
        lax.fori_loop(0, SB // 4, pair, ())
        if jb + 1 < NSB:
            nrow = wrow + (jb + 1) * SB
            pltpu.async_copy(src_hbm.at[pl.ds(nrow, SROWS)], sb[1 - p], isem)
            pltpu.async_copy(dst_hbm.at[pl.ds(nrow, SB)], db[1 - p], isem)
        lax.fori_loop(SB // 4, SB // 2, pair, ())

    pltpu.make_async_copy(h_hbm.at[sidx0.at[0]], rows0, gsem0).wait()
    pltpu.make_async_copy(h_hbm.at[sidx0.at[0]], rows1, gsem1).wait()
    plsc.subcore_barrier()

    pltpu.sync_copy(acc_sh.at[pl.ds(s * RPS, ZCH)], rows0)
    for j in range(NZ):
        r = s * RPS + j * ZCH
        buf = rows0 if j % 2 == 0 else rows1
        nbuf = rows1 if j % 2 == 0 else rows0
        pltpu.async_copy(buf, out_hbm.at[c, pl.ds(r, ZCH)], wsem)
        if j + 1 < NZ:
            pltpu.sync_copy(acc_sh.at[pl.ds(r + ZCH, ZCH)], nbuf)
        pltpu.make_async_copy(buf, out_hbm.at[c, pl.ds(r, ZCH)], wsem).wait()


@jax.jit
def _sc_agg(h, src, dst, zeros):
    mesh = plsc.VectorSubcoreMesh(core_axis_name="c", subcore_axis_name="s")
    return pl.kernel(
        _sc_agg_body,
        out_type=jax.ShapeDtypeStruct((NC, NP, D), jnp.float32),
        mesh=mesh,
        scratch_types=[
            pltpu.VMEM((SROWS, CHUNK), jnp.int32),
            pltpu.VMEM((SROWS, CHUNK), jnp.int32),
            pltpu.VMEM((SB, CHUNK), jnp.int32),
            pltpu.VMEM((SB, CHUNK), jnp.int32),
            pltpu.VMEM((CHUNK, D), jnp.float32),
            pltpu.VMEM((CHUNK, D), jnp.float32),
            pltpu.VMEM_SHARED((NP, D), jnp.float32),
            pltpu.SemaphoreType.DMA,
            pltpu.SemaphoreType.DMA,
            pltpu.SemaphoreType.DMA,
            pltpu.SemaphoreType.DMA,
            pltpu.SemaphoreType.DMA,
            pltpu.SemaphoreType.DMA,
        ],
    )(h, src, dst, zeros)


def _mlp_block(h_ref, a0_ref, a1_ref, w1_ref, b1_ref, w2_ref, b2_ref):
    z = h_ref[...] + a0_ref[0] + a1_ref[0]
    z = jnp.dot(z, w1_ref[...], preferred_element_type=jnp.float32)
    z = jnp.maximum(z + b1_ref[...], 0.0)
    z = jnp.dot(z, w2_ref[...], preferred_element_type=jnp.float32)
    return jnp.maximum(z + b2_ref[...], 0.0)


def _mlp_body(h_ref, a0_ref, a1_ref, w1_ref, b1_ref, w2_ref, b2_ref, out_ref):
    out_ref[...] = _mlp_block(h_ref, a0_ref, a1_ref, w1_ref, b1_ref, w2_ref, b2_ref)


def _mlp_pool_body(h_ref, a0_ref, a1_ref, w1_ref, b1_ref, w2_ref, b2_ref,
                   batch_ref, out_ref, sums, counts):
    i = pl.program_id(0)

    @pl.when(i == 0)
    def _():
        sums[...] = jnp.zeros_like(sums)
        counts[...] = jnp.zeros_like(counts)

    z = _mlp_block(h_ref, a0_ref, a1_ref, w1_ref, b1_ref, w2_ref, b2_ref)
    b = batch_ref[0, 0, :]
    onehot = (b[:, None] == lax.broadcasted_iota(jnp.int32, (1, G), 1)
              ).astype(jnp.float32)
    sums[...] += lax.dot_general(onehot, z, (((0,), (0,)), ((), ())),
                                 preferred_element_type=jnp.float32)
    counts[...] += jnp.broadcast_to(jnp.sum(onehot, axis=0)[:, None], (G, D))

    @pl.when(i == pl.num_programs(0) - 1)
    def _():
        out_ref[...] = sums[...] / jnp.maximum(counts[...], 1.0)


def _mlp_specs():
    return [
        pl.BlockSpec((ROWS, D), lambda i: (i, 0)),
        pl.BlockSpec((1, ROWS, D), lambda i: (0, i, 0)),
        pl.BlockSpec((1, ROWS, D), lambda i: (1, i, 0)),
        pl.BlockSpec((D, D), lambda i: (0, 0)),
        pl.BlockSpec((1, D), lambda i: (0, 0)),
        pl.BlockSpec((D, D), lambda i: (0, 0)),
        pl.BlockSpec((1, D), lambda i: (0, 0)),
    ]


@jax.jit
def _mlp(h, aggs, w1, b1, w2, b2):
    return pl.pallas_call(
        _mlp_body,
        grid=(NB,),
        in_specs=_mlp_specs(),
        out_specs=pl.BlockSpec((ROWS, D), lambda i: (i, 0)),
        out_shape=jax.ShapeDtypeStruct((N, D), jnp.float32),
    )(h, aggs, aggs, w1, b1.reshape(1, D), w2, b2.reshape(1, D))


@jax.jit
def _mlp_pool(h, aggs, w1, b1, w2, b2, batch3):
    return pl.pallas_call(
        _mlp_pool_body,
        grid=(NB,),
        in_specs=_mlp_specs() + [pl.BlockSpec((1, 1, ROWS), lambda i: (i, 0, 0))],
        out_specs=pl.BlockSpec((G, D), lambda i: (0, 0)),
        out_shape=jax.ShapeDtypeStruct((G, D), jnp.float32),
        scratch_shapes=[
            pltpu.VMEM((G, D), jnp.float32),
            pltpu.VMEM((G, D), jnp.float32),
        ],
    )(h, aggs, aggs, w1, b1.reshape(1, D), w2, b2.reshape(1, D), batch3)


def kernel(x, edge_index, batch,
           w1_0, b1_0, w2_0, b2_0,
           w1_1, b1_1, w2_1, b2_1,
           w1_2, b1_2, w2_2, b2_2):
    npad = EP - E
    pad_src = (jnp.arange(npad + 8 * CHUNK, dtype=jnp.int32) * 37) % N
    pad_dst = N + (jnp.arange(npad, dtype=jnp.int32) % (NP - N))
    src = jnp.concatenate([edge_index[0], pad_src]).reshape(-1, CHUNK)
    dst = jnp.concatenate([edge_index[1], pad_dst]).reshape(-1, CHUNK)
    zeros = jnp.zeros((ZCH, D), jnp.float32)
    batch3 = batch.reshape(NB, 1, ROWS)
    params = [(w1_0, b1_0, w2_0, b2_0),
              (w1_1, b1_1, w2_1, b2_1),
              (w1_2, b1_2, w2_2, b2_2)]
    h = x
    for l, (w1, b1, w2, b2) in enumerate(params):
        aggs = _sc_agg(h, src, dst, zeros)
        if l < 2:
            h = _mlp(h, aggs, w1, b1, w2, b2)
        else:
            return _mlp_pool(h, aggs, w1, b1, w2, b2, batch3)


# speedup vs baseline: 1.0320x; 1.0320x over previous
import functools

import jax
import jax.numpy as jnp
from jax import lax
from jax.experimental import pallas as pl
from jax.experimental.pallas import tpu as pltpu
from jax.experimental.pallas import tpu_sc as plsc

N = 10000
E = 320000
D = 128
G = 64

NC = 2
NS = 16
NW = NC * NS
CHUNK = 128
EP = NW * 10240
EPW = EP // NW
NCHUNK = EPW // CHUNK
NP = 10240
RPS = NP // NS
ZCH = 128
NZ = RPS // ZCH

ROWS = 2000
NB = N // ROWS


SB = 16
SROWS = SB + 8
NSB = NCHUNK // SB


def _sc_agg_body(h_hbm, src_hbm, dst_hbm, zeros_hbm, out_hbm,
                 sidx0, sidx1, didx0, didx1, rows0, rows1, acc_sh,
                 isem, gsem0, gsem1, gsem2, gsem3, ssem0, ssem1, wsem):
    c = lax.axis_index("c")
    s = lax.axis_index("s")
    w = s * NC + c
    wrow = w * NCHUNK

    pltpu.sync_copy(zeros_hbm, rows0)
    for j in range(NZ):
        pltpu.async_copy(rows0, acc_sh.at[pl.ds(s * RPS + j * ZCH, ZCH)], wsem)
    pltpu.sync_copy(src_hbm.at[pl.ds(wrow, SROWS)], sidx0)
    pltpu.sync_copy(dst_hbm.at[pl.ds(wrow, SB)], didx0)
    for j in range(NZ):
        pltpu.make_async_copy(rows0, acc_sh.at[pl.ds(0, ZCH)], wsem).wait()
    pltpu.async_copy(h_hbm.at[sidx0.at[0, pl.ds(0, CHUNK // 2)]],
                     rows0.at[pl.ds(0, CHUNK // 2)], gsem0)
    pltpu.async_copy(h_hbm.at[sidx0.at[0, pl.ds(CHUNK // 2, CHUNK // 2)]],
                     rows0.at[pl.ds(CHUNK // 2, CHUNK // 2)], gsem2)
    pltpu.async_copy(h_hbm.at[sidx0.at[1, pl.ds(0, CHUNK // 2)]],
                     rows1.at[pl.ds(0, CHUNK // 2)], gsem1)
    pltpu.async_copy(h_hbm.at[sidx0.at[1, pl.ds(CHUNK // 2, CHUNK // 2)]],
                     rows1.at[pl.ds(CHUNK // 2, CHUNK // 2)], gsem3)
    plsc.subcore_barrier()

    sb = (sidx0, sidx1)
    db = (didx0, didx1)

    HC = CHUNK // 2

    def _g2(buf, sbp, l, sa, sb2):
        pltpu.async_copy(h_hbm.at[sbp.at[l, pl.ds(0, HC)]],
                         buf.at[pl.ds(0, HC)], sa)
        pltpu.async_copy(h_hbm.at[sbp.at[l, pl.ds(HC, HC)]],
                         buf.at[pl.ds(HC, HC)], sb2)

    def _w2(buf, sa, sb2):
        pltpu.make_async_copy(h_hbm.at[sidx0.at[0, pl.ds(0, HC)]],
                              buf.at[pl.ds(0, HC)], sa).wait()
        pltpu.make_async_copy(h_hbm.at[sidx0.at[0, pl.ds(0, HC)]],
                              buf.at[pl.ds(HC, HC)], sb2).wait()

    def _make_pair(sbp, dbp):
        def pair(t, _):
            l0 = 2 * t
            l1 = l0 + 1
            _w2(rows0, gsem0, gsem2)
            pltpu.async_copy(rows0, acc_sh.at[dbp.at[l0]], ssem0, add=True)
            _w2(rows1, gsem1, gsem3)
            pltpu.async_copy(rows1, acc_sh.at[dbp.at[l1]], ssem1, add=True)
            pltpu.make_async_copy(rows0, acc_sh.at[dbp.at[l0]], ssem0).wait()
            _g2(rows0, sbp, l0 + 2, gsem0, gsem2)
            pltpu.make_async_copy(rows1, acc_sh.at[dbp.at[l1]], ssem1).wait()
            _g2(rows1, sbp, l1 + 2, gsem1, gsem3)
            return ()
        return pair

    for jb in range(NSB):
        p = jb % 2
        if jb > 0:
            pltpu.make_async_copy(src_hbm.at[pl.ds(0, SROWS)], sb[p], isem).wait()
            pltpu.make_async_copy(dst_hbm.at[pl.ds(0, SB)], db[p], isem).wait()
        pair = _make_pair(sb[p], db[p])
        lax.fori_loop(0, SB // 4, pair, ())
        if jb + 1 < NSB:
            nrow = wrow + (jb + 1) * SB
            pltpu.async_copy(src_hbm.at[pl.ds(nrow, SROWS)], sb[1 - p], isem)
            pltpu.async_copy(dst_hbm.at[pl.ds(nrow, SB)], db[1 - p], isem)
        lax.fori_loop(SB // 4, SB // 2, pair, ())

    _w2(rows0, gsem0, gsem2)
    _w2(rows1, gsem1, gsem3)
    plsc.subcore_barrier()

    pltpu.sync_copy(acc_sh.at[pl.ds(s * RPS, ZCH)], rows0)
    for j in range(NZ):
        r = s * RPS + j * ZCH
        buf = rows0 if j % 2 == 0 else rows1
        nbuf = rows1 if j % 2 == 0 else rows0
        pltpu.async_copy(buf, out_hbm.at[c, pl.ds(r, ZCH)], wsem)
        if j + 1 < NZ:
            pltpu.sync_copy(acc_sh.at[pl.ds(r + ZCH, ZCH)], nbuf)
        pltpu.make_async_copy(buf, out_hbm.at[c, pl.ds(r, ZCH)], wsem).wait()


@jax.jit
def _sc_agg(h, src, dst, zeros):
    mesh = plsc.VectorSubcoreMesh(core_axis_name="c", subcore_axis_name="s")
    return pl.kernel(
        _sc_agg_body,
        out_type=jax.ShapeDtypeStruct((NC, NP, D), jnp.float32),
        mesh=mesh,
        scratch_types=[
            pltpu.VMEM((SROWS, CHUNK), jnp.int32),
            pltpu.VMEM((SROWS, CHUNK), jnp.int32),
            pltpu.VMEM((SB, CHUNK), jnp.int32),
            pltpu.VMEM((SB, CHUNK), jnp.int32),
            pltpu.VMEM((CHUNK, D), jnp.float32),
            pltpu.VMEM((CHUNK, D), jnp.float32),
            pltpu.VMEM_SHARED((NP, D), jnp.float32),
            pltpu.SemaphoreType.DMA,
            pltpu.SemaphoreType.DMA,
            pltpu.SemaphoreType.DMA,
            pltpu.SemaphoreType.DMA,
            pltpu.SemaphoreType.DMA,
            pltpu.SemaphoreType.DMA,
            pltpu.SemaphoreType.DMA,
            pltpu.SemaphoreType.DMA,
        ],
    )(h, src, dst, zeros)


def _mlp_block(h_ref, a0_ref, a1_ref, w1_ref, b1_ref, w2_ref, b2_ref):
    z = h_ref[...] + a0_ref[0] + a1_ref[0]
    z = jnp.dot(z, w1_ref[...], preferred_element_type=jnp.float32)
    z = jnp.maximum(z + b1_ref[...], 0.0)
    z = jnp.dot(z, w2_ref[...], preferred_element_type=jnp.float32)
    return jnp.maximum(z + b2_ref[...], 0.0)


def _mlp_body(h_ref, a0_ref, a1_ref, w1_ref, b1_ref, w2_ref, b2_ref, out_ref):
    out_ref[...] = _mlp_block(h_ref, a0_ref, a1_ref, w1_ref, b1_ref, w2_ref, b2_ref)


def _mlp_pool_body(h_ref, a0_ref, a1_ref, w1_ref, b1_ref, w2_ref, b2_ref,
                   batch_ref, out_ref, sums, counts):
    i = pl.program_id(0)

    @pl.when(i == 0)
    def _():
        sums[...] = jnp.zeros_like(sums)
        counts[...] = jnp.zeros_like(counts)

    z = _mlp_block(h_ref, a0_ref, a1_ref, w1_ref, b1_ref, w2_ref, b2_ref)
    b = batch_ref[0, 0, :]
    onehot = (b[:, None] == lax.broadcasted_iota(jnp.int32, (1, G), 1)
              ).astype(jnp.float32)
    sums[...] += lax.dot_general(onehot, z, (((0,), (0,)), ((), ())),
                                 preferred_element_type=jnp.float32)
    counts[...] += jnp.broadcast_to(jnp.sum(onehot, axis=0)[:, None], (G, D))

    @pl.when(i == pl.num_programs(0) - 1)
    def _():
        out_ref[...] = sums[...] / jnp.maximum(counts[...], 1.0)


def _mlp_specs():
    return [
        pl.BlockSpec((ROWS, D), lambda i: (i, 0)),
        pl.BlockSpec((1, ROWS, D), lambda i: (0, i, 0)),
        pl.BlockSpec((1, ROWS, D), lambda i: (1, i, 0)),
        pl.BlockSpec((D, D), lambda i: (0, 0)),
        pl.BlockSpec((1, D), lambda i: (0, 0)),
        pl.BlockSpec((D, D), lambda i: (0, 0)),
        pl.BlockSpec((1, D), lambda i: (0, 0)),
    ]


@jax.jit
def _mlp(h, aggs, w1, b1, w2, b2):
    return pl.pallas_call(
        _mlp_body,
        grid=(NB,),
        in_specs=_mlp_specs(),
        out_specs=pl.BlockSpec((ROWS, D), lambda i: (i, 0)),
        out_shape=jax.ShapeDtypeStruct((N, D), jnp.float32),
    )(h, aggs, aggs, w1, b1.reshape(1, D), w2, b2.reshape(1, D))


@jax.jit
def _mlp_pool(h, aggs, w1, b1, w2, b2, batch3):
    return pl.pallas_call(
        _mlp_pool_body,
        grid=(NB,),
        in_specs=_mlp_specs() + [pl.BlockSpec((1, 1, ROWS), lambda i: (i, 0, 0))],
        out_specs=pl.BlockSpec((G, D), lambda i: (0, 0)),
        out_shape=jax.ShapeDtypeStruct((G, D), jnp.float32),
        scratch_shapes=[
            pltpu.VMEM((G, D), jnp.float32),
            pltpu.VMEM((G, D), jnp.float32),
        ],
    )(h, aggs, aggs, w1, b1.reshape(1, D), w2, b2.reshape(1, D), batch3)


def kernel(x, edge_index, batch,
           w1_0, b1_0, w2_0, b2_0,
           w1_1, b1_1, w2_1, b2_1,
           w1_2, b1_2, w2_2, b2_2):
    npad = EP - E
    pad_src = (jnp.arange(npad + 8 * CHUNK, dtype=jnp.int32) * 37) % N
    pad_dst = N + (jnp.arange(npad, dtype=jnp.int32) % (NP - N))
    src = jnp.concatenate([edge_index[0], pad_src]).reshape(-1, CHUNK)
    dst = jnp.concatenate([edge_index[1], pad_dst]).reshape(-1, CHUNK)
    zeros = jnp.zeros((ZCH, D), jnp.float32)
    batch3 = batch.reshape(NB, 1, ROWS)
    params = [(w1_0, b1_0, w2_0, b2_0),
              (w1_1, b1_1, w2_1, b2_1),
              (w1_2, b1_2, w2_2, b2_2)]
    h = x
    for l, (w1, b1, w2, b2) in enumerate(params):
        aggs = _sc_agg(h, src, dst, zeros)
        if l < 2:
            h = _mlp(h, aggs, w1, b1, w2, b2)
        else:
            return _mlp_pool(h, aggs, w1, b1, w2, b2, batch3)
